# baseline (speedup 1.0000x reference)
import functools

import jax
import jax.numpy as jnp
from jax import lax
from jax.experimental import pallas as pl
from jax.experimental.pallas import tpu as pltpu
from jax.experimental.pallas import tpu_sc as plsc

N_NODES = 10000
N_EDGES = 160000
D_IN = 256
D_OUT = 256
D_HALF = D_OUT // 2

NC = 2
NS = 16
CHUNK = 64
N_CHUNKS = 160
NBUF = 2
EDGES_PER_SUB = N_CHUNKS * CHUNK
E_PAD = NS * EDGES_PER_SUB
TRASH_ROW = N_NODES
ACC_ROWS = N_NODES + 8
ROWS_PER_SUB = 624
TAIL_ROWS = N_NODES - NS * ROWS_PER_SUB


def _mm_body(x_ref, w_ref, b_ref, h0_ref, h1_ref):
    h = jnp.dot(x_ref[...], w_ref[...], preferred_element_type=jnp.float32)
    h = h + b_ref[...]
    h0_ref[...] = h[:, :D_HALF]
    h1_ref[...] = h[:, D_HALF:]


def _linear(x, W, b):
    grid = 5
    rows = N_NODES // grid
    return pl.pallas_call(
        _mm_body,
        grid=(grid,),
        in_specs=[
            pl.BlockSpec((rows, D_IN), lambda i: (i, 0)),
            pl.BlockSpec((D_IN, D_OUT), lambda i: (0, 0)),
            pl.BlockSpec((1, D_OUT), lambda i: (0, 0)),
        ],
        out_specs=[
            pl.BlockSpec((rows, D_HALF), lambda i: (i, 0)),
            pl.BlockSpec((rows, D_HALF), lambda i: (i, 0)),
        ],
        out_shape=[
            jax.ShapeDtypeStruct((N_NODES, D_HALF), jnp.float32),
            jax.ShapeDtypeStruct((N_NODES, D_HALF), jnp.float32),
        ],
    )(x, W, b.reshape(1, D_OUT))


def _sc_body(h0, h1, src3, dst3, out, acc,
             src_all, dst_all, rows, gsems):
    cid = lax.axis_index("c")
    sid = lax.axis_index("s")

    def run(h_half, c0):
        r0 = sid * ROWS_PER_SUB
        t0 = NS * ROWS_PER_SUB
        e0 = sid * EDGES_PER_SUB
        pltpu.sync_copy(h_half.at[pl.ds(r0, ROWS_PER_SUB)],
                        acc.at[pl.ds(r0, ROWS_PER_SUB)])

        @pl.when(sid == NS - 1)
        def _():
            pltpu.sync_copy(h_half.at[pl.ds(t0, TAIL_ROWS)],
                            acc.at[pl.ds(t0, TAIL_ROWS)])

        pltpu.sync_copy(src3.at[sid], src_all)
        pltpu.sync_copy(dst3.at[sid], dst_all)
        plsc.subcore_barrier()

        def idx_ref(j):
            return src_all.at[(j // 2), pl.ds((j % 2) * CHUNK, CHUNK)]

        def start_gather(j, k):
            pltpu.async_copy(h_half.at[idx_ref(j)], rows[k], gsems[k])

        def wait_gather(j, k):
            pltpu.make_async_copy(
                h_half.at[idx_ref(j)], rows[k], gsems[k]).wait()

        for k in range(NBUF):
            start_gather(k, k)

        def group(i, carry):
            j0 = i * NBUF
            for k in range(NBUF):
                j = j0 + k
                wait_gather(j, k)
                pltpu.sync_copy(rows[k], acc.at[dst_all.at[j]], add=True)

                @pl.when(j + NBUF < N_CHUNKS)
                def _():
                    start_gather(j + NBUF, k)

            return carry

        lax.fori_loop(0, N_CHUNKS // NBUF, group, 0)
        plsc.subcore_barrier()
        pltpu.sync_copy(acc.at[pl.ds(r0, ROWS_PER_SUB)],
                        out.at[pl.ds(r0, ROWS_PER_SUB), pl.ds(c0, D_HALF)])

        @pl.when(sid == NS - 1)
        def _():
            pltpu.sync_copy(acc.at[pl.ds(t0, TAIL_ROWS)],
                            out.at[pl.ds(t0, TAIL_ROWS), pl.ds(c0, D_HALF)])

    @pl.when(cid == 0)
    def _():
        run(h0, 0)

    @pl.when(cid == 1)
    def _():
        run(h1, D_HALF)


_scatter = functools.partial(
    pl.kernel,
    out_type=jax.ShapeDtypeStruct((N_NODES, D_OUT), jnp.float32),
    mesh=plsc.VectorSubcoreMesh(
        core_axis_name="c", subcore_axis_name="s", num_cores=NC, num_subcores=NS
    ),
    scratch_types=[
        pltpu.VMEM_SHARED((ACC_ROWS, D_HALF), jnp.float32),
        pltpu.VMEM((N_CHUNKS // 2, 2 * CHUNK), jnp.int32),
        pltpu.VMEM((N_CHUNKS, CHUNK), jnp.int32),
        [pltpu.VMEM((CHUNK, D_HALF), jnp.float32)] * NBUF,
        [pltpu.SemaphoreType.DMA] * NBUF,
    ],
)(_sc_body)


def kernel(x, edge_index, W, b):
    src = edge_index[0].astype(jnp.int32)
    dst = edge_index[1].astype(jnp.int32)
    n_pad = E_PAD - N_EDGES
    src = jnp.concatenate([src, jnp.zeros((n_pad,), jnp.int32)])
    dst = jnp.concatenate([dst, jnp.full((n_pad,), TRASH_ROW, jnp.int32)])
    src = src.reshape(NS, N_CHUNKS // 2, 2 * CHUNK)
    dst = dst.reshape(NS, N_CHUNKS, CHUNK)
    h0, h1 = _linear(x, W, b)
    return _scatter(h0, h1, src, dst)


# speedup vs baseline: 1.0943x; 1.0943x over previous
import functools

import jax
import jax.numpy as jnp
from jax import lax
from jax.experimental import pallas as pl
from jax.experimental.pallas import tpu as pltpu
from jax.experimental.pallas import tpu_sc as plsc

N_NODES = 10000
N_EDGES = 160000
D_IN = 256
D_OUT = 256
D_HALF = D_OUT // 2

NC = 2
NS = 16
CHUNK = 64
N_CHUNKS = 160
NBUF = 4
EDGES_PER_SUB = N_CHUNKS * CHUNK
E_PAD = NS * EDGES_PER_SUB
TRASH_ROW = N_NODES
ACC_ROWS = N_NODES + 8
ROWS_PER_SUB = 624
TAIL_ROWS = N_NODES - NS * ROWS_PER_SUB


def _mm_body(x_ref, w_ref, b_ref, h0_ref, h1_ref):
    h = jnp.dot(x_ref[...], w_ref[...], preferred_element_type=jnp.float32)
    h = h + b_ref[...]
    h0_ref[...] = h[:, :D_HALF]
    h1_ref[...] = h[:, D_HALF:]


def _linear(x, W, b):
    grid = 5
    rows = N_NODES // grid
    return pl.pallas_call(
        _mm_body,
        grid=(grid,),
        in_specs=[
            pl.BlockSpec((rows, D_IN), lambda i: (i, 0)),
            pl.BlockSpec((D_IN, D_OUT), lambda i: (0, 0)),
            pl.BlockSpec((1, D_OUT), lambda i: (0, 0)),
        ],
        out_specs=[
            pl.BlockSpec((rows, D_HALF), lambda i: (i, 0)),
            pl.BlockSpec((rows, D_HALF), lambda i: (i, 0)),
        ],
        out_shape=[
            jax.ShapeDtypeStruct((N_NODES, D_HALF), jnp.float32),
            jax.ShapeDtypeStruct((N_NODES, D_HALF), jnp.float32),
        ],
    )(x, W, b.reshape(1, D_OUT))


def _sc_body(h0, h1, src3, dstf, out, acc,
             src_all, rows, dbufs, gsems, dsems):
    cid = lax.axis_index("c")
    sid = lax.axis_index("s")

    def run(h_half, c0):
        r0 = sid * ROWS_PER_SUB
        t0 = NS * ROWS_PER_SUB
        e0 = sid * EDGES_PER_SUB
        pltpu.sync_copy(h_half.at[pl.ds(r0, ROWS_PER_SUB)],
                        acc.at[pl.ds(r0, ROWS_PER_SUB)])

        @pl.when(sid == NS - 1)
        def _():
            pltpu.sync_copy(h_half.at[pl.ds(t0, TAIL_ROWS)],
                            acc.at[pl.ds(t0, TAIL_ROWS)])

        pltpu.sync_copy(src3.at[sid], src_all)
        plsc.subcore_barrier()

        def idx_ref(row, col):
            return src_all.at[row, pl.ds(col * CHUNK, CHUNK)]

        def start_gather(row, col, k):
            pltpu.async_copy(h_half.at[idx_ref(row, col)], rows[k], gsems[k])

        def wait_gather(row, col, k):
            pltpu.make_async_copy(
                h_half.at[idx_ref(row, col)], rows[k], gsems[k]).wait()

        def start_dst(j, k):
            pltpu.async_copy(
                dstf.at[pl.ds(e0 + j * CHUNK, CHUNK)], dbufs[k], dsems[k])

        def wait_dst(j, k):
            pltpu.make_async_copy(
                dstf.at[pl.ds(e0 + j * CHUNK, CHUNK)], dbufs[k], dsems[k]).wait()

        for k in range(NBUF):
            start_gather(k // 2, k % 2, k)
            start_dst(k, k)

        def group(i, carry):
            j0 = i * NBUF
            for k in range(NBUF):
                j = j0 + k
                row = i * 2 + k // 2
                col = k % 2
                wait_gather(row, col, k)
                wait_dst(j, k)
                pltpu.sync_copy(rows[k], acc.at[dbufs[k]], add=True)

                @pl.when(j + NBUF < N_CHUNKS)
                def _():
                    start_gather(row + 2, col, k)
                    start_dst(j + NBUF, k)

            return carry

        lax.fori_loop(0, N_CHUNKS // NBUF, group, 0)
        plsc.subcore_barrier()
        pltpu.sync_copy(acc.at[pl.ds(r0, ROWS_PER_SUB)],
                        out.at[pl.ds(r0, ROWS_PER_SUB), pl.ds(c0, D_HALF)])

        @pl.when(sid == NS - 1)
        def _():
            pltpu.sync_copy(acc.at[pl.ds(t0, TAIL_ROWS)],
                            out.at[pl.ds(t0, TAIL_ROWS), pl.ds(c0, D_HALF)])

    @pl.when(cid == 0)
    def _():
        run(h0, 0)

    @pl.when(cid == 1)
    def _():
        run(h1, D_HALF)


_scatter = functools.partial(
    pl.kernel,
    out_type=jax.ShapeDtypeStruct((N_NODES, D_OUT), jnp.float32),
    mesh=plsc.VectorSubcoreMesh(
        core_axis_name="c", subcore_axis_name="s", num_cores=NC, num_subcores=NS
    ),
    scratch_types=[
        pltpu.VMEM_SHARED((ACC_ROWS, D_HALF), jnp.float32),
        pltpu.VMEM((N_CHUNKS // 2, 2 * CHUNK), jnp.int32),
        [pltpu.VMEM((CHUNK, D_HALF), jnp.float32)] * NBUF,
        [pltpu.VMEM((CHUNK,), jnp.int32)] * NBUF,
        [pltpu.SemaphoreType.DMA] * NBUF,
        [pltpu.SemaphoreType.DMA] * NBUF,
    ],
)(_sc_body)


def kernel(x, edge_index, W, b):
    src = edge_index[0].astype(jnp.int32)
    dst = edge_index[1].astype(jnp.int32)
    n_pad = E_PAD - N_EDGES
    src = jnp.concatenate([src, jnp.zeros((n_pad,), jnp.int32)])
    dst = jnp.concatenate([dst, jnp.full((n_pad,), TRASH_ROW, jnp.int32)])
    src = src.reshape(NS, N_CHUNKS // 2, 2 * CHUNK)
    h0, h1 = _linear(x, W, b)
    return _scatter(h0, h1, src, dst)
